# Initial kernel scaffold; baseline (speedup 1.0000x reference)
import jax
import jax.numpy as jnp
from jax.experimental import pallas as pl


def kernel(features, labels, pred, confidence, queue_mus, queue_sigmas):
    raise NotImplementedError("write your pallas kernel here")


# speedup vs baseline: 1.4672x; 1.4672x over previous
import functools

import jax
import jax.numpy as jnp
from jax.experimental import pallas as pl

T_K = 10.0
_ROW_BLK = 128
_Q_CHUNK = 2048
_PAD_VAL = 1.0e4


def _tc_body(feat_ref, mus_ref, sig_ref, out_ref, *, d, q_pad):
    f = feat_ref[...]
    m = jnp.mean(f, axis=1, keepdims=True)
    c = f - m
    var = jnp.sum(c * c, axis=1, keepdims=True) / (d - 1)
    s = jnp.sqrt(var)

    n_chunks = q_pad // _Q_CHUNK

    def chunk_step(i, best):
        mu = mus_ref[0, pl.ds(i * _Q_CHUNK, _Q_CHUNK)][None, :]
        sg = sig_ref[0, pl.ds(i * _Q_CHUNK, _Q_CHUNK)][None, :]
        dm = m - mu
        ds_ = s - sg
        dist2 = dm * dm + ds_ * ds_
        return jnp.minimum(best, jnp.min(dist2, axis=1, keepdims=True))

    best0 = jnp.full((f.shape[0], 1), jnp.inf, dtype=jnp.float32)
    best = jax.lax.fori_loop(0, n_chunks, chunk_step, best0)
    out_ref[...] = jnp.exp(-T_K * jnp.sqrt(best[:, 0]))


@functools.partial(jax.jit, static_argnames=())
def kernel(features, labels, pred, confidence, queue_mus, queue_sigmas):
    del labels, pred, confidence
    b, d = features.shape
    q = queue_mus.shape[0]
    q_pad = ((q + _Q_CHUNK - 1) // _Q_CHUNK) * _Q_CHUNK
    mus = jnp.pad(queue_mus, (0, q_pad - q), constant_values=_PAD_VAL)[None, :]
    sigs = jnp.pad(queue_sigmas, (0, q_pad - q), constant_values=_PAD_VAL)[None, :]

    grid = (b // _ROW_BLK,)
    out = pl.pallas_call(
        functools.partial(_tc_body, d=d, q_pad=q_pad),
        grid=grid,
        in_specs=[
            pl.BlockSpec((_ROW_BLK, d), lambda i: (i, 0)),
            pl.BlockSpec((1, q_pad), lambda i: (0, 0)),
            pl.BlockSpec((1, q_pad), lambda i: (0, 0)),
        ],
        out_specs=pl.BlockSpec((_ROW_BLK,), lambda i: (i,)),
        out_shape=jax.ShapeDtypeStruct((b,), jnp.float32),
    )(features, mus, sigs)
    return out


# speedup vs baseline: 1.5968x; 1.0883x over previous
import functools

import jax
import jax.numpy as jnp
from jax.experimental import pallas as pl

T_K = 10.0
_ROW_BLK = 128
_Q_CHUNK = 2048
_PAD_VAL = 1.0e4


def _tc_body(feat_ref, mus_ref, sig_ref, out_ref, *, d, q_pad):
    f = feat_ref[...]
    m = jnp.mean(f, axis=1, keepdims=True)
    c = f - m
    var = jnp.sum(c * c, axis=1, keepdims=True) / (d - 1)
    sp = jnp.sqrt(var) - 1.0
    mneg = -2.0 * m
    sneg = -2.0 * sp

    n_chunks = q_pad // _Q_CHUNK

    def chunk_step(i, best):
        mu = mus_ref[0, pl.ds(i * _Q_CHUNK, _Q_CHUNK)][None, :]
        sgp = sig_ref[0, pl.ds(i * _Q_CHUNK, _Q_CHUNK)][None, :] - 1.0
        cq = mu * mu + sgp * sgp
        t = mneg * mu + (sneg * sgp + cq)
        return jnp.minimum(best, jnp.min(t, axis=1, keepdims=True))

    best0 = jnp.full((f.shape[0], 1), jnp.inf, dtype=jnp.float32)
    best = jax.lax.fori_loop(0, n_chunks, chunk_step, best0)
    dist2 = jnp.maximum(best + (m * m + sp * sp), 0.0)
    out_ref[...] = jnp.exp(-T_K * jnp.sqrt(dist2[:, 0]))


@functools.partial(jax.jit, static_argnames=())
def kernel(features, labels, pred, confidence, queue_mus, queue_sigmas):
    del labels, pred, confidence
    b, d = features.shape
    q = queue_mus.shape[0]
    q_pad = ((q + _Q_CHUNK - 1) // _Q_CHUNK) * _Q_CHUNK
    mus = jnp.pad(queue_mus, (0, q_pad - q), constant_values=_PAD_VAL)[None, :]
    sigs = jnp.pad(queue_sigmas, (0, q_pad - q), constant_values=_PAD_VAL)[None, :]

    grid = (b // _ROW_BLK,)
    out = pl.pallas_call(
        functools.partial(_tc_body, d=d, q_pad=q_pad),
        grid=grid,
        in_specs=[
            pl.BlockSpec((_ROW_BLK, d), lambda i: (i, 0)),
            pl.BlockSpec((1, q_pad), lambda i: (0, 0)),
            pl.BlockSpec((1, q_pad), lambda i: (0, 0)),
        ],
        out_specs=pl.BlockSpec((_ROW_BLK,), lambda i: (i,)),
        out_shape=jax.ShapeDtypeStruct((b,), jnp.float32),
    )(features, mus, sigs)
    return out
